# baseline (speedup 1.0000x reference)
import jax
import jax.numpy as jnp
from jax.experimental import pallas as pl
from jax.experimental.pallas import tpu as pltpu

_CHUNK = 2000


def _xwt(x, w):
    return jax.lax.dot_general(
        x, w, (((1,), (1,)), ((), ())), preferred_element_type=jnp.float32)


def _mlp_pipe(x_hbm, w0_ref, b0_ref, w1_ref, b1_ref, w2_ref, b2_ref,
              w3_ref, b3_ref, o_ref, xbuf, sem):
    nch = o_ref.shape[0]
    chunk = xbuf.shape[1]

    def cp(i, slot):
        return pltpu.make_async_copy(
            x_hbm.at[pl.ds(i * chunk, chunk), :], xbuf.at[slot], sem.at[slot])

    cp(0, 0).start()

    def body(i, carry):
        slot = jax.lax.rem(i, 2)

        @pl.when(i + 1 < nch)
        def _():
            cp(i + 1, jax.lax.rem(i + 1, 2)).start()

        cp(i, slot).wait()
        x = xbuf[slot]
        h = jnp.maximum(_xwt(x, w0_ref[...]) + b0_ref[...][None, :], 0.0)
        h = jnp.maximum(_xwt(h, w1_ref[...]) + b1_ref[...][None, :], 0.0)
        h = jnp.maximum(_xwt(h, w2_ref[...]) + b2_ref[...][None, :], 0.0)
        ht = jax.lax.dot_general(
            w3_ref[...], h, (((1,), (1,)), ((), ())),
            preferred_element_type=jnp.float32)
        o_ref[i] = jnp.maximum(ht + b3_ref[...][:, None], 0.0)
        return carry

    jax.lax.fori_loop(0, nch, body, 0)


def kernel(inputs, g, W0, b0, W1, b1, W2, b2, W3, b3):
    del g
    n, in_dim = inputs.shape
    hid = W0.shape[0]
    lat = W3.shape[0]
    nch = n // _CHUNK

    full = lambda shape: pl.BlockSpec(shape, lambda i: (0, 0))
    vec = lambda d: pl.BlockSpec((d,), lambda i: (0,))
    out_t = pl.pallas_call(
        _mlp_pipe,
        grid=(1,),
        in_specs=[
            pl.BlockSpec(memory_space=pl.ANY),
            full((hid, in_dim)), vec(hid),
            full((hid, hid)), vec(hid),
            full((hid, hid)), vec(hid),
            full((lat, hid)), vec(lat),
        ],
        out_specs=pl.BlockSpec((nch, lat, _CHUNK), lambda i: (0, 0, 0)),
        out_shape=jax.ShapeDtypeStruct((nch, lat, _CHUNK), jnp.float32),
        scratch_shapes=[
            pltpu.VMEM((2, _CHUNK, in_dim), jnp.float32),
            pltpu.SemaphoreType.DMA((2,)),
        ],
    )(inputs, W0, b0, W1, b1, W2, b2, W3, b3)
    return out_t.transpose(0, 2, 1).reshape(n, lat)


# speedup vs baseline: 1.0627x; 1.0627x over previous
import jax
import jax.numpy as jnp
from jax.experimental import pallas as pl
from jax.experimental.pallas import tpu as pltpu

_CHUNK = 2000


def _xwt(x, w):
    return jax.lax.dot_general(
        x, w, (((1,), (1,)), ((), ())), preferred_element_type=jnp.float32)


def _mlp_pipe(x_hbm, w0_ref, b0_ref, w1_ref, b1_ref, w2_ref, b2_ref,
              w3_ref, b3_ref, o_ref, xbuf, sem):
    nch = o_ref.shape[0]
    chunk = xbuf.shape[1]

    def cp(i):
        return pltpu.make_async_copy(
            x_hbm.at[pl.ds(i * chunk, chunk), :], xbuf.at[i], sem.at[i])

    for i in range(nch):
        cp(i).start()

    for i in range(nch):
        cp(i).wait()
        x = xbuf[i]
        h = jnp.maximum(_xwt(x, w0_ref[...]) + b0_ref[...][None, :], 0.0)
        h = jnp.maximum(_xwt(h, w1_ref[...]) + b1_ref[...][None, :], 0.0)
        h = jnp.maximum(_xwt(h, w2_ref[...]) + b2_ref[...][None, :], 0.0)
        ht = jax.lax.dot_general(
            w3_ref[...], h, (((1,), (1,)), ((), ())),
            preferred_element_type=jnp.float32)
        o_ref[i] = jnp.maximum(ht + b3_ref[...][:, None], 0.0)


def kernel(inputs, g, W0, b0, W1, b1, W2, b2, W3, b3):
    del g
    n, in_dim = inputs.shape
    hid = W0.shape[0]
    lat = W3.shape[0]
    nch = n // _CHUNK

    full = lambda shape: pl.BlockSpec(shape, lambda i: (0, 0))
    vec = lambda d: pl.BlockSpec((d,), lambda i: (0,))
    out_t = pl.pallas_call(
        _mlp_pipe,
        grid=(1,),
        in_specs=[
            pl.BlockSpec(memory_space=pl.ANY),
            full((hid, in_dim)), vec(hid),
            full((hid, hid)), vec(hid),
            full((hid, hid)), vec(hid),
            full((lat, hid)), vec(lat),
        ],
        out_specs=pl.BlockSpec((nch, lat, _CHUNK), lambda i: (0, 0, 0)),
        out_shape=jax.ShapeDtypeStruct((nch, lat, _CHUNK), jnp.float32),
        scratch_shapes=[
            pltpu.VMEM((nch, _CHUNK, in_dim), jnp.float32),
            pltpu.SemaphoreType.DMA((nch,)),
        ],
    )(inputs, W0, b0, W1, b1, W2, b2, W3, b3)
    return out_t.transpose(0, 2, 1).reshape(n, lat)
